# baseline (speedup 1.0000x reference)
import jax
import jax.numpy as jnp
from jax import lax
from jax.experimental import pallas as pl
from jax.experimental.pallas import tpu as pltpu
from jax.experimental.pallas import tpu_sc as plsc

B, C, H, W = 8, 192, 224, 224
NW = 32
CPW = (B * C) // NW
WPB = C // CPW
HALF = H // 4
NCK = 4 * CPW
NBUF = 8
LOOK = 3


def _shuffle_body(x_hbm, perm_hbm, out_hbm, pbuf, bufs, gsems, ssems):
    cid = lax.axis_index("c")
    sid = lax.axis_index("s")
    wid = sid * 2 + cid
    b = wid // WPB
    cbase = (wid % WPB) * CPW

    pltpu.sync_copy(perm_hbm, pbuf)
    pvs = [pbuf[pl.ds(pl.multiple_of(cbase + 16 * t, 16), 16)]
           for t in range(CPW // 16)]

    def src_c(l):
        return pvs[l // 16][l % 16]

    def fire_gather(i):
        l, h = i // 4, i % 4
        src = x_hbm.at[b, src_c(l), pl.ds(HALF * h, HALF)]
        return pltpu.async_copy(src, bufs[i % NBUF], gsems[i % NBUF])

    def fire_scatter(i):
        l, h = i // 4, i % 4
        dst = out_hbm.at[b, cbase + l, pl.ds(HALF * h, HALF)]
        pltpu.async_copy(bufs[i % NBUF], dst, ssems[i % NBUF])

    def wait_scatter(sem):
        pltpu.make_async_copy(bufs[0], out_hbm.at[0, 0, pl.ds(0, HALF)],
                              sem).wait()

    gds = {i: fire_gather(i) for i in range(LOOK)}
    for i in range(NCK):
        j = i + LOOK
        if j < NCK:
            if j >= NBUF:
                wait_scatter(ssems[j % NBUF])
            gds[j] = fire_gather(j)
        gds.pop(i).wait()
        fire_scatter(i)
    for t in range(NBUF):
        wait_scatter(ssems[t])


@jax.jit
def _shuffle(x, perm):
    mesh = plsc.VectorSubcoreMesh(core_axis_name="c", subcore_axis_name="s")
    return pl.kernel(
        _shuffle_body,
        out_type=jax.ShapeDtypeStruct((B, C, H, W), jnp.float32),
        mesh=mesh,
        compiler_params=pltpu.CompilerParams(use_tc_tiling_on_sc=True),
        scratch_types=[
            pltpu.VMEM((C,), jnp.int32),
            [pltpu.VMEM((HALF, W), jnp.float32) for _ in range(NBUF)],
            [pltpu.SemaphoreType.DMA for _ in range(NBUF)],
            [pltpu.SemaphoreType.DMA for _ in range(NBUF)],
        ],
    )(x, perm)


def kernel(x, perm):
    y = _shuffle(x, perm.astype(jnp.int32))
    return (y, jnp.zeros((), dtype=jnp.float32))


# speedup vs baseline: 1.0107x; 1.0107x over previous
import jax
import jax.numpy as jnp
from jax import lax
from jax.experimental import pallas as pl
from jax.experimental.pallas import tpu as pltpu
from jax.experimental.pallas import tpu_sc as plsc

B, C, H, W = 8, 192, 224, 224
NW = 32
CPW = (B * C) // NW
WPB = C // CPW
HALF = H // 2
NCK = 2 * CPW
NBUF = 4
LOOK = 2


def _shuffle_body(x_hbm, perm_hbm, out_hbm, pbuf, bufs, gsems, ssems):
    cid = lax.axis_index("c")
    sid = lax.axis_index("s")
    wid = sid * 2 + cid
    b = wid // WPB
    cbase = (wid % WPB) * CPW

    pltpu.sync_copy(perm_hbm, pbuf)
    pvs = [pbuf[pl.ds(pl.multiple_of(cbase + 16 * t, 16), 16)]
           for t in range(CPW // 16)]

    def src_c(l):
        return pvs[l // 16][l % 16]

    def fire_gather(i):
        l, h = i // 2, i % 2
        src = x_hbm.at[b, src_c(l), pl.ds(HALF * h, HALF)]
        return pltpu.async_copy(src, bufs[i % NBUF], gsems[i % NBUF])

    def fire_scatter(i):
        l, h = i // 2, i % 2
        dst = out_hbm.at[b, cbase + l, pl.ds(HALF * h, HALF)]
        pltpu.async_copy(bufs[i % NBUF], dst, ssems[i % NBUF])

    def wait_scatter(sem):
        pltpu.make_async_copy(bufs[0], out_hbm.at[0, 0, pl.ds(0, HALF)],
                              sem).wait()

    gds = {i: fire_gather(i) for i in range(LOOK)}
    for i in range(NCK):
        j = i + LOOK
        if j < NCK:
            if j >= NBUF:
                wait_scatter(ssems[j % NBUF])
            gds[j] = fire_gather(j)
        gds.pop(i).wait()
        fire_scatter(i)
    for t in range(NBUF):
        wait_scatter(ssems[t])


@jax.jit
def _shuffle(x, perm):
    mesh = plsc.VectorSubcoreMesh(core_axis_name="c", subcore_axis_name="s")
    return pl.kernel(
        _shuffle_body,
        out_type=jax.ShapeDtypeStruct((B, C, H, W), jnp.float32),
        mesh=mesh,
        compiler_params=pltpu.CompilerParams(use_tc_tiling_on_sc=True),
        scratch_types=[
            pltpu.VMEM((C,), jnp.int32),
            [pltpu.VMEM((HALF, W), jnp.float32) for _ in range(NBUF)],
            [pltpu.SemaphoreType.DMA for _ in range(NBUF)],
            [pltpu.SemaphoreType.DMA for _ in range(NBUF)],
        ],
    )(x, perm)


def kernel(x, perm):
    y = _shuffle(x, perm.astype(jnp.int32))
    return (y, jnp.zeros((), dtype=jnp.float32))
